# baseline (speedup 1.0000x reference)
import jax
import jax.numpy as jnp
from jax.experimental import pallas as pl
from jax.experimental.pallas import tpu as pltpu

SEQ = 200
V = 128
M = 128


def _body(d_ref, t_hbm, w_hbm, b_ref, o_ref, t_v, w_v, sem_t, sem_w):
    ct = pltpu.make_async_copy(t_hbm, t_v, sem_t)
    cw = pltpu.make_async_copy(w_hbm, w_v, sem_w)
    ct.start()
    cw.start()
    d = d_ref[...]
    iota = jax.lax.broadcasted_iota(jnp.int32, (V, SEQ), 0)
    oh = (d == iota).astype(jnp.float32)
    counts = jnp.sum(oh, axis=1, keepdims=True)
    ct.wait()
    cw.wait()
    g = jax.lax.dot_general(t_v[...], w_v[...],
                            (((1,), (1,)), ((), ())),
                            preferred_element_type=jnp.float32)
    out = jnp.sum(counts * g, axis=0, keepdims=True)
    o_ref[...] = out + b_ref[...]


def kernel(data, table, W, b):
    out = pl.pallas_call(
        _body,
        in_specs=[
            pl.BlockSpec(memory_space=pltpu.MemorySpace.VMEM),
            pl.BlockSpec(memory_space=pl.ANY),
            pl.BlockSpec(memory_space=pl.ANY),
            pl.BlockSpec(memory_space=pltpu.MemorySpace.VMEM),
        ],
        scratch_shapes=[
            pltpu.MemorySpace.VMEM((V, 128), jnp.float32),
            pltpu.MemorySpace.VMEM((M, 128), jnp.float32),
            pltpu.SemaphoreType.DMA,
            pltpu.SemaphoreType.DMA,
        ],
        out_shape=jax.ShapeDtypeStruct((1, M), jnp.float32),
    )(data.astype(jnp.int32).reshape(1, SEQ), table, W, b.reshape(1, M))
    return out.reshape(M)


# speedup vs baseline: 1.4345x; 1.4345x over previous
import jax
import jax.numpy as jnp
from jax.experimental import pallas as pl
from jax.experimental.pallas import tpu as pltpu

SEQ = 200
V = 128
M = 128


def _body(d_ref, t_ref, w_ref, b_ref, o_ref):
    d = d_ref[...]
    iota = jax.lax.broadcasted_iota(jnp.int32, (V, SEQ), 0)
    oh = (d == iota).astype(jnp.float32)
    counts = jnp.sum(oh, axis=1, keepdims=True)
    g = jax.lax.dot_general(t_ref[...], w_ref[...],
                            (((1,), (1,)), ((), ())),
                            preferred_element_type=jnp.float32)
    out = jnp.sum(counts * g, axis=0, keepdims=True)
    o_ref[...] = out + b_ref[...]


def kernel(data, table, W, b):
    out = pl.pallas_call(
        _body,
        compiler_params=pltpu.CompilerParams(
            disable_bounds_checks=True,
            disable_semaphore_checks=True,
            skip_device_barrier=True,
        ),
        out_shape=jax.ShapeDtypeStruct((1, M), jnp.float32),
    )(data.astype(jnp.int32).reshape(1, SEQ), table, W, b.reshape(1, M))
    return out.reshape(M)
